# baseline (speedup 1.0000x reference)
import functools

import jax
import jax.numpy as jnp
from jax import lax
from jax.experimental import pallas as pl
from jax.experimental.pallas import tpu as pltpu
from jax.experimental.pallas import tpu_sc as plsc

N = 10000
E = 320000
IN_DIM = 128
HID = 256
OUT = 128

NC = 2
NS = 16
CHUNK = 128
N_PAD = 10240
NW_PAD = 10016
G = 8
E_PAD = 327680
PAD_IDX = N
RPT = N_PAD // NS
RPTW = NW_PAD // NS
ECHUNKS = E_PAD // CHUNK
NBLK = ECHUNKS // G

_MESH = plsc.VectorSubcoreMesh(core_axis_name="c", subcore_axis_name="s")
_SC_PARAMS = pltpu.CompilerParams(use_tc_tiling_on_sc=False,
                                  needs_layout_passes=False)


@functools.partial(
    pl.kernel,
    out_type=jax.ShapeDtypeStruct((NC, N_PAD, 16), jnp.float32),
    mesh=_MESH,
    compiler_params=_SC_PARAMS,
    scratch_types=[
        pltpu.VMEM((ECHUNKS // (NC * NS), CHUNK), jnp.int32),
        pltpu.VMEM((CHUNK, 16), jnp.float32),
        pltpu.VMEM_SHARED((N_PAD, 16), jnp.float32),
    ],
)
def _deg_sc(dst_hbm, ones_hbm, zeros16_hbm, deg_out, dst_v, ones_v, deg_sh):
    cid = lax.axis_index("c")
    sid = lax.axis_index("s")
    wid = cid * NS + sid
    nchunks = ECHUNKS // (NC * NS)
    pltpu.sync_copy(zeros16_hbm, deg_sh.at[pl.ds(sid * RPT, RPT)])
    pltpu.sync_copy(ones_hbm, ones_v)
    pltpu.sync_copy(dst_hbm.at[wid], dst_v)
    plsc.subcore_barrier()

    def body(c, carry):
        pltpu.sync_copy(ones_v, deg_sh.at[dst_v.at[c]], add=True)
        return carry

    lax.fori_loop(0, nchunks, body, None)
    plsc.subcore_barrier()
    pltpu.sync_copy(deg_sh.at[pl.ds(sid * RPT, RPT)],
                    deg_out.at[cid, pl.ds(sid * RPT, RPT)])


_B2 = 1024


def _lin1_body(x_ref, w1_ref, degp_ref, g1_ref, disw_ref):
    deg = degp_ref[0, :, 0:1] + degp_ref[1, :, 0:1] + 1.0
    dis = lax.rsqrt(deg)
    h = jnp.dot(x_ref[...], w1_ref[...], preferred_element_type=jnp.float32)
    g = h * dis
    g1_ref[0] = g[:, :IN_DIM]
    g1_ref[1] = g[:, IN_DIM:]
    disw_ref[...] = jnp.broadcast_to(dis, (_B2, 16))


def _lin1(x_p, W1, degp):
    return pl.pallas_call(
        _lin1_body,
        grid=(N_PAD // _B2,),
        in_specs=[
            pl.BlockSpec((_B2, IN_DIM), lambda i: (i, 0)),
            pl.BlockSpec((IN_DIM, HID), lambda i: (0, 0)),
            pl.BlockSpec((NC, _B2, 16), lambda i: (0, i, 0)),
        ],
        out_specs=[
            pl.BlockSpec((NC, _B2, IN_DIM), lambda i: (0, i, 0)),
            pl.BlockSpec((_B2, 16), lambda i: (i, 0)),
        ],
        out_shape=[
            jax.ShapeDtypeStruct((NC, N_PAD, IN_DIM), jnp.float32),
            jax.ShapeDtypeStruct((N_PAD, 16), jnp.float32),
        ],
    )(x_p, W1, degp)


_NGRP = ECHUNKS // NS // G


@functools.partial(
    pl.kernel,
    out_type=[
        jax.ShapeDtypeStruct((NC, N_PAD, IN_DIM), jnp.float32),
        jax.ShapeDtypeStruct((NC, NW_PAD, 16), jnp.float32),
    ],
    mesh=_MESH,
    compiler_params=_SC_PARAMS,
    scratch_types=[
        pltpu.VMEM((G, CHUNK), jnp.int32),
        pltpu.VMEM((G, CHUNK), jnp.int32),
        pltpu.VMEM((G, CHUNK), jnp.int32),
        pltpu.VMEM((CHUNK, IN_DIM // 2), jnp.int32),
        pltpu.VMEM((CHUNK, IN_DIM // 2), jnp.int32),
        pltpu.VMEM((CHUNK, IN_DIM), jnp.float32),
        pltpu.VMEM((CHUNK, 16), jnp.float32),
        pltpu.SemaphoreType.DMA,
        pltpu.SemaphoreType.DMA,
        pltpu.SemaphoreType.DMA,
        pltpu.VMEM_SHARED((N_PAD, IN_DIM), jnp.float32),
        pltpu.VMEM_SHARED((NW_PAD, 16), jnp.float32),
    ],
)
def _msg_sc(g1flat_hbm, disw_hbm, src2_hbm, dst_hbm, zeros128_hbm,
            zeros16w_hbm, msg_out, w_out,
            srcb, dstb, srcw, gb0, gb1, frows, wrow, sem0, sem1, wsem,
            acc_sh, wacc_sh):
    cid = lax.axis_index("c")
    sid = lax.axis_index("s")
    gbs = (gb0, gb1)
    sems = (sem0, sem1)

    def widen(gb, i, carry):
        for k in range(8):
            r = i * 8 + k
            for q in range(4):
                x = gb[r, pl.ds(q * 16, 16)]
                frows[r, pl.ds(32 * q, 16)] = plsc.bitcast(
                    lax.shift_left(x, 16), jnp.float32)
                frows[r, pl.ds(32 * q + 16, 16)] = plsc.bitcast(
                    lax.bitwise_and(x, jnp.int32(-65536)), jnp.float32)
        return carry

    pltpu.sync_copy(zeros128_hbm, acc_sh.at[pl.ds(sid * RPT, RPT)])
    pltpu.sync_copy(zeros16w_hbm, wacc_sh.at[pl.ds(sid * RPTW, RPTW)])
    plsc.subcore_barrier()

    def group(g, carry):
        pltpu.sync_copy(src2_hbm.at[cid * NBLK + sid * _NGRP + g], srcb)
        pltpu.sync_copy(src2_hbm.at[sid * _NGRP + g], srcw)
        pltpu.sync_copy(dst_hbm.at[sid * _NGRP + g], dstb)
        descs = {0: pltpu.async_copy(g1flat_hbm.at[srcb.at[0]], gb0, sem0)}
        for j in range(G):
            if j + 1 < G:
                descs[j + 1] = pltpu.async_copy(
                    g1flat_hbm.at[srcb.at[j + 1]],
                    gbs[(j + 1) % 2], sems[(j + 1) % 2])
            descs[j].wait()
            lax.fori_loop(0, CHUNK // 8,
                          functools.partial(widen, gbs[j % 2]), None)
            pltpu.sync_copy(frows, acc_sh.at[dstb.at[j]], add=True)

            @pl.when(cid == (j % 2))
            def _():
                pltpu.async_copy(disw_hbm.at[dstb.at[j]], wrow, wsem).wait()
                pltpu.sync_copy(wrow, wacc_sh.at[srcw.at[j]], add=True)

        return carry

    lax.fori_loop(0, _NGRP, group, None)
    plsc.subcore_barrier()
    pltpu.sync_copy(acc_sh.at[pl.ds(sid * RPT, RPT)],
                    msg_out.at[cid, pl.ds(sid * RPT, RPT)])
    pltpu.sync_copy(wacc_sh.at[pl.ds(sid * RPTW, RPTW)],
                    w_out.at[cid, pl.ds(sid * RPTW, RPTW)])


_B4 = 1000


def _out_body(msg_ref, g1p_ref, disw_ref, wp_ref, b1_ref, w2_ref, b2_ref,
              out_ref):
    i = pl.program_id(0)
    dis = disw_ref[:, 0:1]
    wreal = wp_ref[0, :, 0:1] + wp_ref[1, :, 0:1]
    r0 = jnp.maximum((msg_ref[0] + g1p_ref[0]) * dis + b1_ref[0:1, :], 0.0)
    r1 = jnp.maximum((msg_ref[1] + g1p_ref[1]) * dis + b1_ref[1:2, :], 0.0)
    h2 = (jnp.dot(r0, w2_ref[0], preferred_element_type=jnp.float32)
          + jnp.dot(r1, w2_ref[1], preferred_element_type=jnp.float32))
    wd = (wreal + dis) * dis
    contrib = lax.dot_general(wd, h2, (((0,), (0,)), ((), ())),
                              preferred_element_type=jnp.float32)

    @pl.when(i == 0)
    def _():
        out_ref[...] = jnp.zeros_like(out_ref)

    out_ref[...] += contrib

    @pl.when(i == (N // _B4) - 1)
    def _():
        out_ref[...] = out_ref[...] * (1.0 / N) + b2_ref[...]


def _lin2(msg, g1p, disw, wp, b1h, W2h, b2r):
    return pl.pallas_call(
        _out_body,
        grid=(N // _B4,),
        in_specs=[
            pl.BlockSpec((NC, _B4, IN_DIM), lambda i: (0, i, 0)),
            pl.BlockSpec((NC, _B4, IN_DIM), lambda i: (0, i, 0)),
            pl.BlockSpec((_B4, 16), lambda i: (i, 0)),
            pl.BlockSpec((NC, _B4, 16), lambda i: (0, i, 0)),
            pl.BlockSpec((2, IN_DIM), lambda i: (0, 0)),
            pl.BlockSpec((2, IN_DIM, OUT), lambda i: (0, 0, 0)),
            pl.BlockSpec((1, OUT), lambda i: (0, 0)),
        ],
        out_specs=pl.BlockSpec((1, OUT), lambda i: (0, 0)),
        out_shape=jax.ShapeDtypeStruct((1, OUT), jnp.float32),
    )(msg, g1p, disw, wp, b1h, W2h, b2r)


def kernel(x, edge_index, W1, b1, W2, b2):
    src = edge_index[0].astype(jnp.int32)
    dst = edge_index[1].astype(jnp.int32)
    fill = jnp.full((E_PAD - E,), PAD_IDX, jnp.int32)
    src_p = jnp.concatenate([src, fill])
    dst_p = jnp.concatenate([dst, fill])
    dst_k1 = dst_p.reshape(NC * NS, ECHUNKS // (NC * NS), CHUNK)
    dst_b = dst_p.reshape(NBLK, G, CHUNK)
    src2 = jnp.concatenate([src_p, src_p + N_PAD]).reshape(2 * NBLK, G, CHUNK)
    x_p = jnp.pad(x, ((0, N_PAD - N), (0, 0)))
    ones16 = jnp.ones((CHUNK, 16), jnp.float32)
    z16 = jnp.zeros((RPT, 16), jnp.float32)
    z16w = jnp.zeros((RPTW, 16), jnp.float32)
    z128 = jnp.zeros((RPT, IN_DIM), jnp.float32)

    degp = _deg_sc(dst_k1, ones16, z16)
    g1p, disw = _lin1(x_p, W1, degp)
    gperm = (g1p.reshape(NC, N_PAD, 4, 2, 16).swapaxes(3, 4)
             .reshape(NC * N_PAD, IN_DIM))
    tbl = jax.lax.bitcast_convert_type(
        gperm.astype(jnp.bfloat16).reshape(NC * N_PAD, IN_DIM // 2, 2),
        jnp.int32)
    msg, wp = _msg_sc(tbl, disw, src2, dst_b, z128, z16w)
    out = _lin2(msg, g1p, disw, wp, b1.reshape(2, IN_DIM),
                W2.reshape(2, IN_DIM, OUT), b2.reshape(1, OUT))
    return out.reshape(OUT)


# speedup vs baseline: 1.1040x; 1.1040x over previous
import functools

import jax
import jax.numpy as jnp
from jax import lax
from jax.experimental import pallas as pl
from jax.experimental.pallas import tpu as pltpu
from jax.experimental.pallas import tpu_sc as plsc

N = 10000
E = 320000
IN_DIM = 128
HID = 256
OUT = 128

NC = 2
NS = 16
CHUNK = 128
N_PAD = 10240
NW_PAD = 10016
G = 8
E_PAD = 327680
PAD_IDX = N
RPT = N_PAD // NS
RPTW = NW_PAD // NS
ECHUNKS = E_PAD // CHUNK
NBLK = ECHUNKS // G

_MESH = plsc.VectorSubcoreMesh(core_axis_name="c", subcore_axis_name="s")
_SC_PARAMS = pltpu.CompilerParams(use_tc_tiling_on_sc=False,
                                  needs_layout_passes=False)


@functools.partial(
    pl.kernel,
    out_type=jax.ShapeDtypeStruct((NC, N_PAD, 16), jnp.float32),
    mesh=_MESH,
    compiler_params=_SC_PARAMS,
    scratch_types=[
        pltpu.VMEM((ECHUNKS // (NC * NS), CHUNK), jnp.int32),
        pltpu.VMEM((CHUNK, 16), jnp.float32),
        pltpu.VMEM_SHARED((N_PAD, 16), jnp.float32),
    ],
)
def _deg_sc(dst_hbm, ones_hbm, zeros16_hbm, deg_out, dst_v, ones_v, deg_sh):
    cid = lax.axis_index("c")
    sid = lax.axis_index("s")
    wid = cid * NS + sid
    nchunks = ECHUNKS // (NC * NS)
    pltpu.sync_copy(zeros16_hbm, deg_sh.at[pl.ds(sid * RPT, RPT)])
    pltpu.sync_copy(ones_hbm, ones_v)
    pltpu.sync_copy(dst_hbm.at[wid], dst_v)
    plsc.subcore_barrier()

    def body(c, carry):
        pltpu.sync_copy(ones_v, deg_sh.at[dst_v.at[c]], add=True)
        return carry

    lax.fori_loop(0, nchunks, body, None)
    plsc.subcore_barrier()
    pltpu.sync_copy(deg_sh.at[pl.ds(sid * RPT, RPT)],
                    deg_out.at[cid, pl.ds(sid * RPT, RPT)])


_B2 = 1024


def _lin1_body(x_ref, w1_ref, degp_ref, g1_ref, disw_ref):
    deg = degp_ref[0, :, 0:1] + degp_ref[1, :, 0:1] + 1.0
    dis = lax.rsqrt(deg)
    h = jnp.dot(x_ref[...], w1_ref[...], preferred_element_type=jnp.float32)
    g = h * dis
    g1_ref[0] = g[:, :IN_DIM]
    g1_ref[1] = g[:, IN_DIM:]
    disw_ref[...] = jnp.broadcast_to(dis, (_B2, 16))


def _lin1(x_p, W1, degp):
    return pl.pallas_call(
        _lin1_body,
        grid=(N_PAD // _B2,),
        in_specs=[
            pl.BlockSpec((_B2, IN_DIM), lambda i: (i, 0)),
            pl.BlockSpec((IN_DIM, HID), lambda i: (0, 0)),
            pl.BlockSpec((NC, _B2, 16), lambda i: (0, i, 0)),
        ],
        out_specs=[
            pl.BlockSpec((NC, _B2, IN_DIM), lambda i: (0, i, 0)),
            pl.BlockSpec((_B2, 16), lambda i: (i, 0)),
        ],
        out_shape=[
            jax.ShapeDtypeStruct((NC, N_PAD, IN_DIM), jnp.float32),
            jax.ShapeDtypeStruct((N_PAD, 16), jnp.float32),
        ],
    )(x_p, W1, degp)


_NGRP = ECHUNKS // NS // G


@functools.partial(
    pl.kernel,
    out_type=[
        jax.ShapeDtypeStruct((NC, N_PAD, IN_DIM), jnp.float32),
        jax.ShapeDtypeStruct((NC, NW_PAD, 16), jnp.float32),
    ],
    mesh=_MESH,
    compiler_params=_SC_PARAMS,
    scratch_types=[
        pltpu.VMEM((G, CHUNK), jnp.int32),
        pltpu.VMEM((G, CHUNK), jnp.int32),
        pltpu.VMEM((G, CHUNK), jnp.int32),
        pltpu.VMEM((CHUNK, IN_DIM // 2), jnp.int32),
        pltpu.VMEM((CHUNK, IN_DIM // 2), jnp.int32),
        pltpu.VMEM((CHUNK, IN_DIM), jnp.float32),
        pltpu.VMEM((CHUNK, 16), jnp.float32),
        pltpu.SemaphoreType.DMA,
        pltpu.SemaphoreType.DMA,
        pltpu.SemaphoreType.DMA,
        pltpu.SemaphoreType.DMA,
        pltpu.SemaphoreType.DMA,
        pltpu.VMEM_SHARED((N_PAD, IN_DIM), jnp.float32),
        pltpu.VMEM_SHARED((NW_PAD, 16), jnp.float32),
    ],
)
def _msg_sc(g1flat_hbm, disw_hbm, src2_hbm, dst_hbm, zeros128_hbm,
            zeros16w_hbm, msg_out, w_out,
            srcb, dstb, srcw, gb0, gb1, frows, wrow, sem0, sem1, wsem,
            sema, semb, acc_sh, wacc_sh):
    cid = lax.axis_index("c")
    sid = lax.axis_index("s")
    gbs = (gb0, gb1)
    sems = (sem0, sem1)

    def widen(gb, i, carry):
        for k in range(8):
            r = i * 8 + k
            for q in range(4):
                x = gb[r, pl.ds(q * 16, 16)]
                frows[r, pl.ds(32 * q, 16)] = plsc.bitcast(
                    lax.shift_left(x, 16), jnp.float32)
                frows[r, pl.ds(32 * q + 16, 16)] = plsc.bitcast(
                    lax.bitwise_and(x, jnp.int32(-65536)), jnp.float32)
        return carry

    pltpu.sync_copy(zeros128_hbm, acc_sh.at[pl.ds(sid * RPT, RPT)])
    pltpu.sync_copy(zeros16w_hbm, wacc_sh.at[pl.ds(sid * RPTW, RPTW)])
    plsc.subcore_barrier()

    def group(g, carry):
        pltpu.sync_copy(src2_hbm.at[cid * NBLK + sid * _NGRP + g], srcb)
        pltpu.sync_copy(src2_hbm.at[sid * _NGRP + g], srcw)
        pltpu.sync_copy(dst_hbm.at[sid * _NGRP + g], dstb)
        H = CHUNK // 2
        HB = CHUNK // 16
        descs = {0: pltpu.async_copy(g1flat_hbm.at[srcb.at[0]], gb0, sem0)}
        sca = scb = None
        for j in range(G):
            if j + 1 < G:
                descs[j + 1] = pltpu.async_copy(
                    g1flat_hbm.at[srcb.at[j + 1]],
                    gbs[(j + 1) % 2], sems[(j + 1) % 2])
            descs[j].wait()
            if sca is not None:
                sca.wait()
            lax.fori_loop(0, HB, functools.partial(widen, gbs[j % 2]), None)
            sca = pltpu.async_copy(frows.at[pl.ds(0, H)],
                                   acc_sh.at[dstb.at[j, pl.ds(0, H)]],
                                   sema, add=True)
            if scb is not None:
                scb.wait()
            lax.fori_loop(HB, 2 * HB,
                          functools.partial(widen, gbs[j % 2]), None)
            scb = pltpu.async_copy(frows.at[pl.ds(H, H)],
                                   acc_sh.at[dstb.at[j, pl.ds(H, H)]],
                                   semb, add=True)

            @pl.when(cid == (j % 2))
            def _():
                pltpu.async_copy(disw_hbm.at[dstb.at[j]], wrow, wsem).wait()
                pltpu.sync_copy(wrow, wacc_sh.at[srcw.at[j]], add=True)

        sca.wait()
        scb.wait()
        return carry

    lax.fori_loop(0, _NGRP, group, None)
    plsc.subcore_barrier()
    pltpu.sync_copy(acc_sh.at[pl.ds(sid * RPT, RPT)],
                    msg_out.at[cid, pl.ds(sid * RPT, RPT)])
    pltpu.sync_copy(wacc_sh.at[pl.ds(sid * RPTW, RPTW)],
                    w_out.at[cid, pl.ds(sid * RPTW, RPTW)])


_B4 = 1000


def _out_body(msg_ref, g1p_ref, disw_ref, wp_ref, b1_ref, w2_ref, b2_ref,
              out_ref):
    i = pl.program_id(0)
    dis = disw_ref[:, 0:1]
    wreal = wp_ref[0, :, 0:1] + wp_ref[1, :, 0:1]
    r0 = jnp.maximum((msg_ref[0] + g1p_ref[0]) * dis + b1_ref[0:1, :], 0.0)
    r1 = jnp.maximum((msg_ref[1] + g1p_ref[1]) * dis + b1_ref[1:2, :], 0.0)
    h2 = (jnp.dot(r0, w2_ref[0], preferred_element_type=jnp.float32)
          + jnp.dot(r1, w2_ref[1], preferred_element_type=jnp.float32))
    wd = (wreal + dis) * dis
    contrib = lax.dot_general(wd, h2, (((0,), (0,)), ((), ())),
                              preferred_element_type=jnp.float32)

    @pl.when(i == 0)
    def _():
        out_ref[...] = jnp.zeros_like(out_ref)

    out_ref[...] += contrib

    @pl.when(i == (N // _B4) - 1)
    def _():
        out_ref[...] = out_ref[...] * (1.0 / N) + b2_ref[...]


def _lin2(msg, g1p, disw, wp, b1h, W2h, b2r):
    return pl.pallas_call(
        _out_body,
        grid=(N // _B4,),
        in_specs=[
            pl.BlockSpec((NC, _B4, IN_DIM), lambda i: (0, i, 0)),
            pl.BlockSpec((NC, _B4, IN_DIM), lambda i: (0, i, 0)),
            pl.BlockSpec((_B4, 16), lambda i: (i, 0)),
            pl.BlockSpec((NC, _B4, 16), lambda i: (0, i, 0)),
            pl.BlockSpec((2, IN_DIM), lambda i: (0, 0)),
            pl.BlockSpec((2, IN_DIM, OUT), lambda i: (0, 0, 0)),
            pl.BlockSpec((1, OUT), lambda i: (0, 0)),
        ],
        out_specs=pl.BlockSpec((1, OUT), lambda i: (0, 0)),
        out_shape=jax.ShapeDtypeStruct((1, OUT), jnp.float32),
    )(msg, g1p, disw, wp, b1h, W2h, b2r)


def kernel(x, edge_index, W1, b1, W2, b2):
    src = edge_index[0].astype(jnp.int32)
    dst = edge_index[1].astype(jnp.int32)
    fill = jnp.full((E_PAD - E,), PAD_IDX, jnp.int32)
    src_p = jnp.concatenate([src, fill])
    dst_p = jnp.concatenate([dst, fill])
    dst_k1 = dst_p.reshape(NC * NS, ECHUNKS // (NC * NS), CHUNK)
    dst_b = dst_p.reshape(NBLK, G, CHUNK)
    src2 = jnp.concatenate([src_p, src_p + N_PAD]).reshape(2 * NBLK, G, CHUNK)
    x_p = jnp.pad(x, ((0, N_PAD - N), (0, 0)))
    ones16 = jnp.ones((CHUNK, 16), jnp.float32)
    z16 = jnp.zeros((RPT, 16), jnp.float32)
    z16w = jnp.zeros((RPTW, 16), jnp.float32)
    z128 = jnp.zeros((RPT, IN_DIM), jnp.float32)

    degp = _deg_sc(dst_k1, ones16, z16)
    g1p, disw = _lin1(x_p, W1, degp)
    gperm = (g1p.reshape(NC, N_PAD, 4, 2, 16).swapaxes(3, 4)
             .reshape(NC * N_PAD, IN_DIM))
    tbl = jax.lax.bitcast_convert_type(
        gperm.astype(jnp.bfloat16).reshape(NC * N_PAD, IN_DIM // 2, 2),
        jnp.int32)
    msg, wp = _msg_sc(tbl, disw, src2, dst_b, z128, z16w)
    out = _lin2(msg, g1p, disw, wp, b1.reshape(2, IN_DIM),
                W2.reshape(2, IN_DIM, OUT), b2.reshape(1, OUT))
    return out.reshape(OUT)
